# baseline (speedup 1.0000x reference)
import functools

import jax
import jax.numpy as jnp
from jax import lax
from jax.experimental import pallas as pl
from jax.experimental.pallas import tpu as pltpu
from jax.experimental.pallas import tpu_sc as plsc

NUM_ENTITIES = 1000000
NUM_RELATIONS = 1000
DIM = 64
PADW = 128
BATCH = 16384

NC = 2
NS = 16
NW = NC * NS
B_PER_W = BATCH // NW
HALF = B_PER_W // 2
CHUNK = 128
RBLK = 512


def _relayout_body(x_ref, o_ref):
    xt = jnp.transpose(x_ref[...], (1, 0))
    o_ref[...] = jnp.concatenate(
        [xt, jnp.zeros((RBLK, PADW - DIM), jnp.float32)], axis=1)


def _relayout_tc(ent_t):
    return pl.pallas_call(
        _relayout_body,
        grid=(pl.cdiv(NUM_ENTITIES, RBLK),),
        in_specs=[pl.BlockSpec((DIM, RBLK), lambda c: (0, c))],
        out_specs=pl.BlockSpec((RBLK, PADW), lambda c: (c, 0)),
        out_shape=jax.ShapeDtypeStruct((NUM_ENTITIES, PADW), jnp.float32),
    )(ent_t)


def _sc_body(h_idx_hbm, r_idx_hbm, t_idx_hbm, ent_hbm, rel_hbm, out_hbm,
             hidx_v, ridx_v, tidx_v, h_v, r_v, t_v, m_v, out_v,
             sem_h, sem_r, sem_t):
    wid = lax.axis_index("s") * NC + lax.axis_index("c")
    base = wid * B_PER_W

    pltpu.sync_copy(h_idx_hbm.at[pl.ds(base, B_PER_W)], hidx_v)
    pltpu.sync_copy(r_idx_hbm.at[pl.ds(base, B_PER_W)], ridx_v)
    pltpu.sync_copy(t_idx_hbm.at[pl.ds(base, B_PER_W)], tidx_v)

    lanes = lax.iota(jnp.int32, 16)

    def _sqrt16(x):
        xs = jnp.maximum(x, jnp.float32(1e-30))
        i = plsc.bitcast(xs, jnp.int32)
        i = jnp.int32(0x5F3759DF) - lax.shift_right_arithmetic(i, jnp.int32(1))
        y = plsc.bitcast(i, jnp.float32)
        half = jnp.float32(0.5) * xs
        for _ in range(3):
            y = y * (jnp.float32(1.5) - half * y * y)
        return xs * y

    for hp in range(2):
        offs = hp * HALF
        copies = []
        for j in range(HALF // CHUNK):
            isl = pl.ds(offs + j * CHUNK, CHUNK)
            dsl = pl.ds(j * CHUNK, CHUNK)
            copies.append(
                pltpu.async_copy(ent_hbm.at[hidx_v.at[isl]], h_v.at[dsl],
                                 sem_h))
            copies.append(
                pltpu.async_copy(rel_hbm.at[ridx_v.at[isl]], r_v.at[dsl],
                                 sem_r))
            copies.append(
                pltpu.async_copy(ent_hbm.at[tidx_v.at[isl]], t_v.at[dsl],
                                 sem_t))
        for c in copies:
            c.wait()

        def block_body(i, carry):
            b0 = i * 16
            for row in range(16):
                b = b0 + row
                acc = jnp.zeros((16,), jnp.float32)
                for s in range(DIM // 16):
                    sl = pl.ds(s * 16, 16)
                    d = (h_v[b, sl] + r_v[b, sl]) - t_v[b, sl]
                    acc = acc + d * d
                m_v[row, pl.ds(0, 16)] = acc
            tot = jnp.zeros((16,), jnp.float32)
            for j in range(16):
                col = plsc.load_gather(
                    m_v, [lanes, jnp.full((16,), j, jnp.int32)])
                tot = tot + col
            out_v[pl.ds(offs + b0, 16)] = _sqrt16(tot)
            return carry

        lax.fori_loop(0, HALF // 16, block_body, 0)

    pltpu.sync_copy(out_v, out_hbm.at[pl.ds(base, B_PER_W)])


@jax.jit
def _transe_sc(h_idx, r_idx, t_idx, entity_emb, rel_emb):
    ent = _relayout_tc(entity_emb.T)
    mesh = plsc.VectorSubcoreMesh(core_axis_name="c", subcore_axis_name="s")
    return pl.kernel(
        _sc_body,
        out_type=jax.ShapeDtypeStruct((BATCH,), jnp.float32),
        mesh=mesh,
        compiler_params=pltpu.CompilerParams(
            needs_layout_passes=False, use_tc_tiling_on_sc=False),
        scratch_types=[
            pltpu.VMEM((B_PER_W,), jnp.int32),
            pltpu.VMEM((B_PER_W,), jnp.int32),
            pltpu.VMEM((B_PER_W,), jnp.int32),
            pltpu.VMEM((HALF, PADW), jnp.float32),
            pltpu.VMEM((HALF, DIM), jnp.float32),
            pltpu.VMEM((HALF, PADW), jnp.float32),
            pltpu.VMEM((16, 17), jnp.float32),
            pltpu.VMEM((B_PER_W,), jnp.float32),
            pltpu.SemaphoreType.DMA,
            pltpu.SemaphoreType.DMA,
            pltpu.SemaphoreType.DMA,
        ],
    )(h_idx, r_idx, t_idx, ent, rel_emb)


def kernel(h_idx, r_idx, t_idx, entity_emb, rel_emb):
    return _transe_sc(h_idx.astype(jnp.int32), r_idx.astype(jnp.int32),
                      t_idx.astype(jnp.int32), entity_emb, rel_emb)


# speedup vs baseline: 2.1734x; 2.1734x over previous
import functools

import jax
import jax.numpy as jnp
from jax import lax
from jax.experimental import pallas as pl
from jax.experimental.pallas import tpu as pltpu
from jax.experimental.pallas import tpu_sc as plsc

NUM_ENTITIES = 1000000
NUM_RELATIONS = 1000
DIM = 64
PADW = 128
BATCH = 16384

NC = 2
NS = 16
NW = NC * NS
B_PER_W = BATCH // NW
HALF = B_PER_W // 2
CHUNK = 128
RBLK = 512


def _sc_body(h_idx_hbm, r_idx_hbm, t_idx_hbm, ent_hbm, rel_hbm, out_hbm,
             hidx_v, ridx_v, tidx_v, h_v, r_v, t_v, m_v, out_v,
             sem_h, sem_r, sem_t):
    wid = lax.axis_index("s") * NC + lax.axis_index("c")
    base = wid * B_PER_W

    pltpu.sync_copy(h_idx_hbm.at[pl.ds(base, B_PER_W)], hidx_v)
    pltpu.sync_copy(r_idx_hbm.at[pl.ds(base, B_PER_W)], ridx_v)
    pltpu.sync_copy(t_idx_hbm.at[pl.ds(base, B_PER_W)], tidx_v)

    lanes = lax.iota(jnp.int32, 16)

    def _sqrt16(x):
        xs = jnp.maximum(x, jnp.float32(1e-30))
        i = plsc.bitcast(xs, jnp.int32)
        i = jnp.int32(0x5F3759DF) - lax.shift_right_arithmetic(i, jnp.int32(1))
        y = plsc.bitcast(i, jnp.float32)
        half = jnp.float32(0.5) * xs
        for _ in range(3):
            y = y * (jnp.float32(1.5) - half * y * y)
        return xs * y

    for hp in range(2):
        offs = hp * HALF
        copies = []
        for j in range(HALF // CHUNK):
            isl = pl.ds(offs + j * CHUNK, CHUNK)
            dsl = pl.ds(j * CHUNK, CHUNK)
            copies.append(
                pltpu.async_copy(ent_hbm.at[hidx_v.at[isl]], h_v.at[dsl],
                                 sem_h))
            copies.append(
                pltpu.async_copy(rel_hbm.at[ridx_v.at[isl]], r_v.at[dsl],
                                 sem_r))
            copies.append(
                pltpu.async_copy(ent_hbm.at[tidx_v.at[isl]], t_v.at[dsl],
                                 sem_t))
        for c in copies:
            c.wait()

        def block_body(i, carry):
            b0 = i * 16
            for row in range(16):
                b = b0 + row
                acc = jnp.zeros((16,), jnp.float32)
                for s in range(DIM // 16):
                    sl = pl.ds(s * 16, 16)
                    d = (h_v[b, sl] + r_v[b, sl]) - t_v[b, sl]
                    acc = acc + d * d
                m_v[row, pl.ds(0, 16)] = acc
            tot = jnp.zeros((16,), jnp.float32)
            for j in range(16):
                col = plsc.load_gather(
                    m_v, [lanes, jnp.full((16,), j, jnp.int32)])
                tot = tot + col
            out_v[pl.ds(offs + b0, 16)] = _sqrt16(tot)
            return carry

        lax.fori_loop(0, HALF // 16, block_body, 0)

    pltpu.sync_copy(out_v, out_hbm.at[pl.ds(base, B_PER_W)])


@jax.jit
def _transe_sc(h_idx, r_idx, t_idx, entity_emb, rel_emb):
    ent = jnp.pad(entity_emb, ((0, 0), (0, PADW - DIM)))
    mesh = plsc.VectorSubcoreMesh(core_axis_name="c", subcore_axis_name="s")
    return pl.kernel(
        _sc_body,
        out_type=jax.ShapeDtypeStruct((BATCH,), jnp.float32),
        mesh=mesh,
        compiler_params=pltpu.CompilerParams(
            needs_layout_passes=False, use_tc_tiling_on_sc=False),
        scratch_types=[
            pltpu.VMEM((B_PER_W,), jnp.int32),
            pltpu.VMEM((B_PER_W,), jnp.int32),
            pltpu.VMEM((B_PER_W,), jnp.int32),
            pltpu.VMEM((HALF, PADW), jnp.float32),
            pltpu.VMEM((HALF, DIM), jnp.float32),
            pltpu.VMEM((HALF, PADW), jnp.float32),
            pltpu.VMEM((16, 17), jnp.float32),
            pltpu.VMEM((B_PER_W,), jnp.float32),
            pltpu.SemaphoreType.DMA,
            pltpu.SemaphoreType.DMA,
            pltpu.SemaphoreType.DMA,
        ],
    )(h_idx, r_idx, t_idx, ent, rel_emb)


def kernel(h_idx, r_idx, t_idx, entity_emb, rel_emb):
    return _transe_sc(h_idx.astype(jnp.int32), r_idx.astype(jnp.int32),
                      t_idx.astype(jnp.int32), entity_emb, rel_emb)


# speedup vs baseline: 2.4754x; 1.1390x over previous
import functools

import jax
import jax.numpy as jnp
from jax import lax
from jax.experimental import pallas as pl
from jax.experimental.pallas import tpu as pltpu
from jax.experimental.pallas import tpu_sc as plsc

NUM_ENTITIES = 1000000
NUM_RELATIONS = 1000
DIM = 64
PADW = 128
BATCH = 16384

NC = 2
NS = 16
NW = NC * NS
B_PER_W = BATCH // NW
HALF = B_PER_W // 2
CHUNK = 128


RBLK = 2048


def _relayout_body(x_ref, o_ref):
    xt = jnp.transpose(x_ref[...], (1, 0))
    o_ref[...] = jnp.concatenate(
        [xt, jnp.zeros((RBLK, PADW - DIM), jnp.float32)], axis=1)


def _relayout_tc(ent_t):
    return pl.pallas_call(
        _relayout_body,
        grid=(pl.cdiv(NUM_ENTITIES, RBLK),),
        in_specs=[pl.BlockSpec((DIM, RBLK), lambda c: (0, c))],
        out_specs=pl.BlockSpec((RBLK, PADW), lambda c: (c, 0)),
        out_shape=jax.ShapeDtypeStruct((NUM_ENTITIES, PADW), jnp.float32),
    )(ent_t)


def _sc_body(h_idx_hbm, r_idx_hbm, t_idx_hbm, ent_hbm, rel_hbm, out_hbm,
             hidx_v, ridx_v, tidx_v, h_v, r_v, t_v, m_v, out_v,
             sem_h, sem_r, sem_t):
    wid = lax.axis_index("s") * NC + lax.axis_index("c")
    base = wid * B_PER_W

    pltpu.sync_copy(h_idx_hbm.at[pl.ds(base, B_PER_W)], hidx_v)
    pltpu.sync_copy(r_idx_hbm.at[pl.ds(base, B_PER_W)], ridx_v)
    pltpu.sync_copy(t_idx_hbm.at[pl.ds(base, B_PER_W)], tidx_v)

    lanes = lax.iota(jnp.int32, 16)

    def _sqrt16(x):
        xs = jnp.maximum(x, jnp.float32(1e-30))
        i = plsc.bitcast(xs, jnp.int32)
        i = jnp.int32(0x5F3759DF) - lax.shift_right_arithmetic(i, jnp.int32(1))
        y = plsc.bitcast(i, jnp.float32)
        half = jnp.float32(0.5) * xs
        for _ in range(3):
            y = y * (jnp.float32(1.5) - half * y * y)
        return xs * y

    for hp in range(2):
        offs = hp * HALF
        copies = []
        for j in range(HALF // CHUNK):
            isl = pl.ds(offs + j * CHUNK, CHUNK)
            dsl = pl.ds(j * CHUNK, CHUNK)
            copies.append(
                pltpu.async_copy(ent_hbm.at[hidx_v.at[isl]], h_v.at[dsl],
                                 sem_h))
            copies.append(
                pltpu.async_copy(rel_hbm.at[ridx_v.at[isl]], r_v.at[dsl],
                                 sem_r))
            copies.append(
                pltpu.async_copy(ent_hbm.at[tidx_v.at[isl]], t_v.at[dsl],
                                 sem_t))
        for c in copies:
            c.wait()

        def block_body(i, carry):
            b0 = i * 16
            for row in range(16):
                b = b0 + row
                acc = jnp.zeros((16,), jnp.float32)
                for s in range(DIM // 16):
                    sl = pl.ds(s * 16, 16)
                    d = (h_v[b, sl] + r_v[b, sl]) - t_v[b, sl]
                    acc = acc + d * d
                m_v[row, pl.ds(0, 16)] = acc
            tot = jnp.zeros((16,), jnp.float32)
            for j in range(16):
                col = plsc.load_gather(
                    m_v, [lanes, jnp.full((16,), j, jnp.int32)])
                tot = tot + col
            out_v[pl.ds(offs + b0, 16)] = _sqrt16(tot)
            return carry

        lax.fori_loop(0, HALF // 16, block_body, 0)

    pltpu.sync_copy(out_v, out_hbm.at[pl.ds(base, B_PER_W)])


@jax.jit
def _transe_sc(h_idx, r_idx, t_idx, entity_emb, rel_emb):
    ent = _relayout_tc(entity_emb.T)
    mesh = plsc.VectorSubcoreMesh(core_axis_name="c", subcore_axis_name="s")
    return pl.kernel(
        _sc_body,
        out_type=jax.ShapeDtypeStruct((BATCH,), jnp.float32),
        mesh=mesh,
        compiler_params=pltpu.CompilerParams(
            needs_layout_passes=False, use_tc_tiling_on_sc=False),
        scratch_types=[
            pltpu.VMEM((B_PER_W,), jnp.int32),
            pltpu.VMEM((B_PER_W,), jnp.int32),
            pltpu.VMEM((B_PER_W,), jnp.int32),
            pltpu.VMEM((HALF, PADW), jnp.float32),
            pltpu.VMEM((HALF, DIM), jnp.float32),
            pltpu.VMEM((HALF, PADW), jnp.float32),
            pltpu.VMEM((16, 17), jnp.float32),
            pltpu.VMEM((B_PER_W,), jnp.float32),
            pltpu.SemaphoreType.DMA,
            pltpu.SemaphoreType.DMA,
            pltpu.SemaphoreType.DMA,
        ],
    )(h_idx, r_idx, t_idx, ent, rel_emb)


def kernel(h_idx, r_idx, t_idx, entity_emb, rel_emb):
    return _transe_sc(h_idx.astype(jnp.int32), r_idx.astype(jnp.int32),
                      t_idx.astype(jnp.int32), entity_emb, rel_emb)


# speedup vs baseline: 4.0922x; 1.6531x over previous
import functools

import jax
import jax.numpy as jnp
from jax import lax
from jax.experimental import pallas as pl
from jax.experimental.pallas import tpu as pltpu
from jax.experimental.pallas import tpu_sc as plsc

NUM_ENTITIES = 1000000
NUM_RELATIONS = 1000
DIM = 64
PADW = 128
BATCH = 16384

NC = 2
NS = 16
NW = NC * NS
B_PER_W = BATCH // NW
HALF = B_PER_W // 2
CHUNK = 128


RBLK = 8192


def _relayout_body(x_ref, o_ref):
    xt = jnp.transpose(x_ref[...], (1, 0))
    o_ref[...] = jnp.concatenate(
        [xt, jnp.zeros((RBLK, PADW - DIM), jnp.float32)], axis=1)


def _relayout_tc(ent_t):
    return pl.pallas_call(
        _relayout_body,
        grid=(pl.cdiv(NUM_ENTITIES, RBLK),),
        in_specs=[pl.BlockSpec((DIM, RBLK), lambda c: (0, c))],
        out_specs=pl.BlockSpec((RBLK, PADW), lambda c: (c, 0)),
        out_shape=jax.ShapeDtypeStruct((NUM_ENTITIES, PADW), jnp.float32),
    )(ent_t)


def _sc_body(h_idx_hbm, r_idx_hbm, t_idx_hbm, ent_hbm, rel_hbm, out_hbm,
             hidx_v, ridx_v, tidx_v, h_v, r_v, t_v, m_v, out_v,
             sem_h, sem_r, sem_t):
    wid = lax.axis_index("s") * NC + lax.axis_index("c")
    base = wid * B_PER_W

    pltpu.sync_copy(h_idx_hbm.at[pl.ds(base, B_PER_W)], hidx_v)
    pltpu.sync_copy(r_idx_hbm.at[pl.ds(base, B_PER_W)], ridx_v)
    pltpu.sync_copy(t_idx_hbm.at[pl.ds(base, B_PER_W)], tidx_v)

    lanes = lax.iota(jnp.int32, 16)

    def _sqrt16(x):
        xs = jnp.maximum(x, jnp.float32(1e-30))
        i = plsc.bitcast(xs, jnp.int32)
        i = jnp.int32(0x5F3759DF) - lax.shift_right_arithmetic(i, jnp.int32(1))
        y = plsc.bitcast(i, jnp.float32)
        half = jnp.float32(0.5) * xs
        for _ in range(3):
            y = y * (jnp.float32(1.5) - half * y * y)
        return xs * y

    for hp in range(2):
        offs = hp * HALF
        copies = []
        for j in range(HALF // CHUNK):
            isl = pl.ds(offs + j * CHUNK, CHUNK)
            dsl = pl.ds(j * CHUNK, CHUNK)
            copies.append(
                pltpu.async_copy(ent_hbm.at[hidx_v.at[isl]], h_v.at[dsl],
                                 sem_h))
            copies.append(
                pltpu.async_copy(rel_hbm.at[ridx_v.at[isl]], r_v.at[dsl],
                                 sem_r))
            copies.append(
                pltpu.async_copy(ent_hbm.at[tidx_v.at[isl]], t_v.at[dsl],
                                 sem_t))
        for c in copies:
            c.wait()

        def block_body(i, carry):
            b0 = i * 16
            for row in range(16):
                b = b0 + row
                acc = jnp.zeros((16,), jnp.float32)
                for s in range(DIM // 16):
                    sl = pl.ds(s * 16, 16)
                    d = (h_v[b, sl] + r_v[b, sl]) - t_v[b, sl]
                    acc = acc + d * d
                m_v[row, pl.ds(0, 16)] = acc
            tot = jnp.zeros((16,), jnp.float32)
            for j in range(16):
                col = plsc.load_gather(
                    m_v, [lanes, jnp.full((16,), j, jnp.int32)])
                tot = tot + col
            out_v[pl.ds(offs + b0, 16)] = _sqrt16(tot)
            return carry

        lax.fori_loop(0, HALF // 16, block_body, 0)

    pltpu.sync_copy(out_v, out_hbm.at[pl.ds(base, B_PER_W)])


@jax.jit
def _transe_sc(h_idx, r_idx, t_idx, entity_emb, rel_emb):
    ent = _relayout_tc(entity_emb.T)
    mesh = plsc.VectorSubcoreMesh(core_axis_name="c", subcore_axis_name="s")
    return pl.kernel(
        _sc_body,
        out_type=jax.ShapeDtypeStruct((BATCH,), jnp.float32),
        mesh=mesh,
        compiler_params=pltpu.CompilerParams(
            needs_layout_passes=False, use_tc_tiling_on_sc=False),
        scratch_types=[
            pltpu.VMEM((B_PER_W,), jnp.int32),
            pltpu.VMEM((B_PER_W,), jnp.int32),
            pltpu.VMEM((B_PER_W,), jnp.int32),
            pltpu.VMEM((HALF, PADW), jnp.float32),
            pltpu.VMEM((HALF, DIM), jnp.float32),
            pltpu.VMEM((HALF, PADW), jnp.float32),
            pltpu.VMEM((16, 17), jnp.float32),
            pltpu.VMEM((B_PER_W,), jnp.float32),
            pltpu.SemaphoreType.DMA,
            pltpu.SemaphoreType.DMA,
            pltpu.SemaphoreType.DMA,
        ],
    )(h_idx, r_idx, t_idx, ent, rel_emb)


def kernel(h_idx, r_idx, t_idx, entity_emb, rel_emb):
    return _transe_sc(h_idx.astype(jnp.int32), r_idx.astype(jnp.int32),
                      t_idx.astype(jnp.int32), entity_emb, rel_emb)


# speedup vs baseline: 4.3927x; 1.0734x over previous
import functools

import jax
import jax.numpy as jnp
from jax import lax
from jax.experimental import pallas as pl
from jax.experimental.pallas import tpu as pltpu
from jax.experimental.pallas import tpu_sc as plsc

NUM_ENTITIES = 1000000
NUM_RELATIONS = 1000
DIM = 64
PADW = 128
BATCH = 16384

NC = 2
NS = 16
NW = NC * NS
B_PER_W = BATCH // NW
HALF = B_PER_W // 2
CHUNK = 128


RBLK = 16384


def _relayout_body(x_ref, o_ref):
    xt = jnp.transpose(x_ref[...], (1, 0))
    o_ref[...] = jnp.concatenate(
        [xt, jnp.zeros((RBLK, PADW - DIM), jnp.float32)], axis=1)


def _relayout_tc(ent_t):
    return pl.pallas_call(
        _relayout_body,
        grid=(pl.cdiv(NUM_ENTITIES, RBLK),),
        in_specs=[pl.BlockSpec((DIM, RBLK), lambda c: (0, c))],
        out_specs=pl.BlockSpec((RBLK, PADW), lambda c: (c, 0)),
        out_shape=jax.ShapeDtypeStruct((NUM_ENTITIES, PADW), jnp.float32),
    )(ent_t)


def _sc_body(h_idx_hbm, r_idx_hbm, t_idx_hbm, ent_hbm, rel_hbm, out_hbm,
             hidx_v, ridx_v, tidx_v, h_v, r_v, t_v, m_v, out_v,
             sem_h, sem_r, sem_t):
    wid = lax.axis_index("s") * NC + lax.axis_index("c")
    base = wid * B_PER_W

    pltpu.sync_copy(h_idx_hbm.at[pl.ds(base, B_PER_W)], hidx_v)
    pltpu.sync_copy(r_idx_hbm.at[pl.ds(base, B_PER_W)], ridx_v)
    pltpu.sync_copy(t_idx_hbm.at[pl.ds(base, B_PER_W)], tidx_v)

    lanes = lax.iota(jnp.int32, 16)

    def _sqrt16(x):
        xs = jnp.maximum(x, jnp.float32(1e-30))
        i = plsc.bitcast(xs, jnp.int32)
        i = jnp.int32(0x5F3759DF) - lax.shift_right_arithmetic(i, jnp.int32(1))
        y = plsc.bitcast(i, jnp.float32)
        half = jnp.float32(0.5) * xs
        for _ in range(3):
            y = y * (jnp.float32(1.5) - half * y * y)
        return xs * y

    for hp in range(2):
        offs = hp * HALF
        copies = []
        for j in range(HALF // CHUNK):
            isl = pl.ds(offs + j * CHUNK, CHUNK)
            dsl = pl.ds(j * CHUNK, CHUNK)
            copies.append(
                pltpu.async_copy(ent_hbm.at[hidx_v.at[isl]], h_v.at[dsl],
                                 sem_h))
            copies.append(
                pltpu.async_copy(rel_hbm.at[ridx_v.at[isl]], r_v.at[dsl],
                                 sem_r))
            copies.append(
                pltpu.async_copy(ent_hbm.at[tidx_v.at[isl]], t_v.at[dsl],
                                 sem_t))
        for c in copies:
            c.wait()

        def block_body(i, carry):
            b0 = i * 16
            for row in range(16):
                b = b0 + row
                acc = jnp.zeros((16,), jnp.float32)
                for s in range(DIM // 16):
                    sl = pl.ds(s * 16, 16)
                    d = (h_v[b, sl] + r_v[b, sl]) - t_v[b, sl]
                    acc = acc + d * d
                m_v[row, pl.ds(0, 16)] = acc
            tot = jnp.zeros((16,), jnp.float32)
            for j in range(16):
                col = plsc.load_gather(
                    m_v, [lanes, jnp.full((16,), j, jnp.int32)])
                tot = tot + col
            out_v[pl.ds(offs + b0, 16)] = _sqrt16(tot)
            return carry

        lax.fori_loop(0, HALF // 16, block_body, 0)

    pltpu.sync_copy(out_v, out_hbm.at[pl.ds(base, B_PER_W)])


@jax.jit
def _transe_sc(h_idx, r_idx, t_idx, entity_emb, rel_emb):
    ent = _relayout_tc(entity_emb.T)
    mesh = plsc.VectorSubcoreMesh(core_axis_name="c", subcore_axis_name="s")
    return pl.kernel(
        _sc_body,
        out_type=jax.ShapeDtypeStruct((BATCH,), jnp.float32),
        mesh=mesh,
        compiler_params=pltpu.CompilerParams(
            needs_layout_passes=False, use_tc_tiling_on_sc=False),
        scratch_types=[
            pltpu.VMEM((B_PER_W,), jnp.int32),
            pltpu.VMEM((B_PER_W,), jnp.int32),
            pltpu.VMEM((B_PER_W,), jnp.int32),
            pltpu.VMEM((HALF, PADW), jnp.float32),
            pltpu.VMEM((HALF, DIM), jnp.float32),
            pltpu.VMEM((HALF, PADW), jnp.float32),
            pltpu.VMEM((16, 17), jnp.float32),
            pltpu.VMEM((B_PER_W,), jnp.float32),
            pltpu.SemaphoreType.DMA,
            pltpu.SemaphoreType.DMA,
            pltpu.SemaphoreType.DMA,
        ],
    )(h_idx, r_idx, t_idx, ent, rel_emb)


def kernel(h_idx, r_idx, t_idx, entity_emb, rel_emb):
    return _transe_sc(h_idx.astype(jnp.int32), r_idx.astype(jnp.int32),
                      t_idx.astype(jnp.int32), entity_emb, rel_emb)


# speedup vs baseline: 4.4822x; 1.0204x over previous
import functools

import jax
import jax.numpy as jnp
from jax import lax
from jax.experimental import pallas as pl
from jax.experimental.pallas import tpu as pltpu
from jax.experimental.pallas import tpu_sc as plsc

NUM_ENTITIES = 1000000
NUM_RELATIONS = 1000
DIM = 64
PADW = 128
BATCH = 16384

NC = 2
NS = 16
NW = NC * NS
B_PER_W = BATCH // NW
HALF = B_PER_W // 2
CHUNK = 128


RBLK = 32768


def _relayout_body(x_ref, o_ref):
    xt = jnp.transpose(x_ref[...], (1, 0))
    o_ref[...] = jnp.concatenate(
        [xt, jnp.zeros((RBLK, PADW - DIM), jnp.float32)], axis=1)


def _relayout_tc(ent_t):
    return pl.pallas_call(
        _relayout_body,
        grid=(pl.cdiv(NUM_ENTITIES, RBLK),),
        in_specs=[pl.BlockSpec((DIM, RBLK), lambda c: (0, c))],
        out_specs=pl.BlockSpec((RBLK, PADW), lambda c: (c, 0)),
        out_shape=jax.ShapeDtypeStruct((NUM_ENTITIES, PADW), jnp.float32),
    )(ent_t)


def _sc_body(h_idx_hbm, r_idx_hbm, t_idx_hbm, ent_hbm, rel_hbm, out_hbm,
             hidx_v, ridx_v, tidx_v, h_v, r_v, t_v, m_v, out_v,
             sem_h, sem_r, sem_t):
    wid = lax.axis_index("s") * NC + lax.axis_index("c")
    base = wid * B_PER_W

    pltpu.sync_copy(h_idx_hbm.at[pl.ds(base, B_PER_W)], hidx_v)
    pltpu.sync_copy(r_idx_hbm.at[pl.ds(base, B_PER_W)], ridx_v)
    pltpu.sync_copy(t_idx_hbm.at[pl.ds(base, B_PER_W)], tidx_v)

    lanes = lax.iota(jnp.int32, 16)

    def _sqrt16(x):
        xs = jnp.maximum(x, jnp.float32(1e-30))
        i = plsc.bitcast(xs, jnp.int32)
        i = jnp.int32(0x5F3759DF) - lax.shift_right_arithmetic(i, jnp.int32(1))
        y = plsc.bitcast(i, jnp.float32)
        half = jnp.float32(0.5) * xs
        for _ in range(3):
            y = y * (jnp.float32(1.5) - half * y * y)
        return xs * y

    for hp in range(2):
        offs = hp * HALF
        copies = []
        for j in range(HALF // CHUNK):
            isl = pl.ds(offs + j * CHUNK, CHUNK)
            dsl = pl.ds(j * CHUNK, CHUNK)
            copies.append(
                pltpu.async_copy(ent_hbm.at[hidx_v.at[isl]], h_v.at[dsl],
                                 sem_h))
            copies.append(
                pltpu.async_copy(rel_hbm.at[ridx_v.at[isl]], r_v.at[dsl],
                                 sem_r))
            copies.append(
                pltpu.async_copy(ent_hbm.at[tidx_v.at[isl]], t_v.at[dsl],
                                 sem_t))
        for c in copies:
            c.wait()

        def block_body(i, carry):
            b0 = i * 16
            for row in range(16):
                b = b0 + row
                acc = jnp.zeros((16,), jnp.float32)
                for s in range(DIM // 16):
                    sl = pl.ds(s * 16, 16)
                    d = (h_v[b, sl] + r_v[b, sl]) - t_v[b, sl]
                    acc = acc + d * d
                m_v[row, pl.ds(0, 16)] = acc
            tot = jnp.zeros((16,), jnp.float32)
            for j in range(16):
                col = plsc.load_gather(
                    m_v, [lanes, jnp.full((16,), j, jnp.int32)])
                tot = tot + col
            out_v[pl.ds(offs + b0, 16)] = _sqrt16(tot)
            return carry

        lax.fori_loop(0, HALF // 16, block_body, 0)

    pltpu.sync_copy(out_v, out_hbm.at[pl.ds(base, B_PER_W)])


@jax.jit
def _transe_sc(h_idx, r_idx, t_idx, entity_emb, rel_emb):
    ent = _relayout_tc(entity_emb.T)
    mesh = plsc.VectorSubcoreMesh(core_axis_name="c", subcore_axis_name="s")
    return pl.kernel(
        _sc_body,
        out_type=jax.ShapeDtypeStruct((BATCH,), jnp.float32),
        mesh=mesh,
        compiler_params=pltpu.CompilerParams(
            needs_layout_passes=False, use_tc_tiling_on_sc=False),
        scratch_types=[
            pltpu.VMEM((B_PER_W,), jnp.int32),
            pltpu.VMEM((B_PER_W,), jnp.int32),
            pltpu.VMEM((B_PER_W,), jnp.int32),
            pltpu.VMEM((HALF, PADW), jnp.float32),
            pltpu.VMEM((HALF, DIM), jnp.float32),
            pltpu.VMEM((HALF, PADW), jnp.float32),
            pltpu.VMEM((16, 17), jnp.float32),
            pltpu.VMEM((B_PER_W,), jnp.float32),
            pltpu.SemaphoreType.DMA,
            pltpu.SemaphoreType.DMA,
            pltpu.SemaphoreType.DMA,
        ],
    )(h_idx, r_idx, t_idx, ent, rel_emb)


def kernel(h_idx, r_idx, t_idx, entity_emb, rel_emb):
    return _transe_sc(h_idx.astype(jnp.int32), r_idx.astype(jnp.int32),
                      t_idx.astype(jnp.int32), entity_emb, rel_emb)
